# Initial kernel scaffold; baseline (speedup 1.0000x reference)
import jax
import jax.numpy as jnp
from jax.experimental import pallas as pl


def kernel(x, Wg, W1, b1, W2, b2):
    raise NotImplementedError("write your pallas kernel here")


# speedup vs baseline: 1.0435x; 1.0435x over previous
import functools

import jax
import jax.numpy as jnp
from jax import lax
from jax.experimental import pallas as pl
from jax.experimental.pallas import tpu as pltpu
from jax.experimental.pallas import tpu_sc as plsc

B, T, D, E, F, K = 1, 2048, 1024, 8, 2048, 2
S = B * T
N = S * K
TM = 256
NT = N // TM
U = NT + E - 1
TMR = 512
TMP = 256

SC_NC, SC_NS = 2, 16
NW = SC_NC * SC_NS
GCH = 32


def _router_body(l_ref, topi_ref, gates_ref):
    logits = l_ref[...]
    idx = lax.broadcasted_iota(jnp.int32, logits.shape, 1)
    m1 = jnp.max(logits, axis=1, keepdims=True)
    i1 = jnp.min(jnp.where(logits == m1, idx, E), axis=1, keepdims=True)
    l2 = jnp.where(idx == i1, -jnp.inf, logits)
    m2 = jnp.max(l2, axis=1, keepdims=True)
    i2 = jnp.min(jnp.where(l2 == m2, idx, E), axis=1, keepdims=True)
    e2 = jnp.exp(m2 - m1)
    denom = 1.0 + e2
    topi_ref[...] = jnp.concatenate([i1, i2], axis=1)
    gates_ref[...] = jnp.concatenate([1.0 / denom, e2 / denom], axis=1)


def _router(logits):
    return pl.pallas_call(
        _router_body,
        grid=(S // TMR,),
        in_specs=[
            pl.BlockSpec((TMR, E), lambda i: (i, 0)),
        ],
        out_specs=[
            pl.BlockSpec((TMR, K), lambda i: (i, 0)),
            pl.BlockSpec((TMR, K), lambda i: (i, 0)),
        ],
        out_shape=[
            jax.ShapeDtypeStruct((S, K), jnp.int32),
            jax.ShapeDtypeStruct((S, K), jnp.float32),
        ],
    )(logits)


def _route_metadata(topi, gates):
    e_flat = topi.reshape(N).astype(jnp.int32)
    g_flat = gates.reshape(N)
    oneh = (e_flat[:, None] == jnp.arange(E, dtype=jnp.int32)[None, :])
    csum = jnp.cumsum(oneh.astype(jnp.int32), axis=0)
    counts = csum[-1]
    off = jnp.concatenate(
        [jnp.zeros((1,), jnp.int32), jnp.cumsum(counts)]).astype(jnp.int32)
    within = jnp.take_along_axis(csum, e_flat[:, None], axis=1)[:, 0] - 1
    pos = off[e_flat] + within
    sort_idx = jnp.zeros((N,), jnp.int32).at[pos].set(
        jnp.arange(N, dtype=jnp.int32))
    tok_sorted = sort_idx // K
    g_sorted = g_flat[sort_idx][:, None]

    t_lo = off[:E] // TM
    t_hi = (off[1:] + TM - 1) // TM
    n_e = jnp.where(counts > 0, t_hi - t_lo, 0).astype(jnp.int32)
    cumu = jnp.concatenate(
        [jnp.zeros((1,), jnp.int32), jnp.cumsum(n_e)]).astype(jnp.int32)
    V = cumu[E]
    v = jnp.arange(U, dtype=jnp.int32)
    ev = jnp.clip(jnp.searchsorted(cumu, v, side="right").astype(jnp.int32) - 1,
                  0, E - 1)
    e_last = ev[jnp.maximum(V - 1, 0)]
    real = v < V
    ev = jnp.where(real, ev, e_last)
    ut = jnp.where(real, t_lo[ev] + (v - cumu[ev]), NT - 1)
    us = jnp.where(real, jnp.maximum(off[ev], ut * TM), 0)
    ue = jnp.where(real, jnp.minimum(off[ev + 1], (ut + 1) * TM), 0)
    uf = jnp.concatenate(
        [jnp.ones((1,), jnp.int32), (ut[1:] != ut[:-1]).astype(jnp.int32)])
    return tok_sorted, g_sorted, pos, ev, ut, us, ue, uf


def _sc_gather_body(n_rows, table_hbm, idx_hbm, out_hbm, idx_v, rows_v, sem):
    bpw = n_rows // NW
    wid = lax.axis_index("s") * SC_NC + lax.axis_index("c")
    base = wid * bpw
    pltpu.sync_copy(idx_hbm.at[pl.ds(base, bpw)], idx_v)

    def chunk(i, carry):
        pltpu.async_copy(
            table_hbm.at[idx_v.at[pl.ds(i * GCH, GCH)]], rows_v, sem).wait()
        pltpu.sync_copy(rows_v, out_hbm.at[pl.ds(base + i * GCH, GCH)])
        return carry

    lax.fori_loop(0, bpw // GCH, chunk, 0)


def _sc_gather(table, idx, n_rows):
    mesh = plsc.VectorSubcoreMesh(core_axis_name="c", subcore_axis_name="s",
                                  num_cores=SC_NC, num_subcores=SC_NS)
    return pl.kernel(
        functools.partial(_sc_gather_body, n_rows),
        out_type=jax.ShapeDtypeStruct((n_rows, D), jnp.float32),
        mesh=mesh,
        scratch_types=[
            pltpu.VMEM((n_rows // NW,), jnp.int32),
            pltpu.VMEM((GCH, D), jnp.float32),
            pltpu.SemaphoreType.DMA,
        ],
    )(table, idx)


def _ffn_body(ev_r, ut_r, us_r, ue_r, uf_r,
              x_ref, g_ref, w1_ref, b1_ref, w2_ref, b2_ref, out_ref):
    u = pl.program_id(0)
    xb = x_ref[...].astype(jnp.bfloat16)
    h = jnp.dot(xb, w1_ref[0], preferred_element_type=jnp.float32)
    h = h + b1_ref[0, 0][None, :]
    a = jnp.maximum(h, 0.0).astype(jnp.bfloat16)
    o = jnp.dot(a, w2_ref[0], preferred_element_type=jnp.float32)
    o = o + b2_ref[0, 0][None, :]
    rows = ut_r[u] * TM + lax.broadcasted_iota(jnp.int32, (TM, 1), 0)
    m = (rows >= us_r[u]) & (rows < ue_r[u])
    contrib = jnp.where(m, o * g_ref[...], 0.0)

    @pl.when(uf_r[u] == 1)
    def _init():
        out_ref[...] = contrib

    @pl.when(uf_r[u] == 0)
    def _accum():
        out_ref[...] = out_ref[...] + contrib


def _grouped_ffn(x_sorted, g_sorted, W1b, b1r, W2b, b2r, ev, ut, us, ue, uf):
    grid_spec = pltpu.PrefetchScalarGridSpec(
        num_scalar_prefetch=5,
        grid=(U,),
        in_specs=[
            pl.BlockSpec((TM, D), lambda u, ev, ut, us, ue, uf: (ut[u], 0)),
            pl.BlockSpec((TM, 1), lambda u, ev, ut, us, ue, uf: (ut[u], 0)),
            pl.BlockSpec((1, D, F), lambda u, ev, ut, us, ue, uf: (ev[u], 0, 0)),
            pl.BlockSpec((1, 1, F), lambda u, ev, ut, us, ue, uf: (ev[u], 0, 0)),
            pl.BlockSpec((1, F, D), lambda u, ev, ut, us, ue, uf: (ev[u], 0, 0)),
            pl.BlockSpec((1, 1, D), lambda u, ev, ut, us, ue, uf: (ev[u], 0, 0)),
        ],
        out_specs=pl.BlockSpec((TM, D), lambda u, ev, ut, us, ue, uf: (ut[u], 0)),
    )
    return pl.pallas_call(
        _ffn_body,
        grid_spec=grid_spec,
        out_shape=jax.ShapeDtypeStruct((N, D), jnp.float32),
        compiler_params=pltpu.CompilerParams(
            dimension_semantics=("arbitrary",)),
    )(ev, ut, us, ue, uf, x_sorted, g_sorted, W1b, b1r, W2b, b2r)


def _pairsum_body(a_ref, b_ref, y_ref):
    y_ref[...] = a_ref[...] + b_ref[...]


def _pairsum(o_a, o_b):
    return pl.pallas_call(
        _pairsum_body,
        grid=(S // TMP,),
        in_specs=[
            pl.BlockSpec((TMP, D), lambda i: (i, 0)),
            pl.BlockSpec((TMP, D), lambda i: (i, 0)),
        ],
        out_specs=pl.BlockSpec((TMP, D), lambda i: (i, 0)),
        out_shape=jax.ShapeDtypeStruct((S, D), jnp.float32),
    )(o_a, o_b)


def kernel(x, Wg, W1, b1, W2, b2):
    xf = x.reshape(S, D)
    logits = xf @ Wg
    topi, gates = _router(logits)
    tok_sorted, g_sorted, pos, ev, ut, us, ue, uf = _route_metadata(topi, gates)

    x_sorted = _sc_gather(xf, tok_sorted, N)

    W1b = W1.astype(jnp.bfloat16)
    W2b = W2.astype(jnp.bfloat16)
    b1r = b1[:, None, :]
    b2r = b2[:, None, :]
    o_sorted = _grouped_ffn(x_sorted, g_sorted, W1b, b1r, W2b, b2r,
                            ev, ut, us, ue, uf)

    o_a = _sc_gather(o_sorted, pos[0::2], S)
    o_b = _sc_gather(o_sorted, pos[1::2], S)
    y = _pairsum(o_a, o_b)
    return y.reshape(B, T, D)


# speedup vs baseline: 1.1165x; 1.0700x over previous
import functools

import jax
import jax.numpy as jnp
from jax import lax
from jax.experimental import pallas as pl
from jax.experimental.pallas import tpu as pltpu
from jax.experimental.pallas import tpu_sc as plsc

B, T, D, E, F, K = 1, 2048, 1024, 8, 2048, 2
S = B * T
N = S * K
TM = 256
NT = N // TM
U = NT + E - 1
TMR = 512
TMP = 256

SC_NC, SC_NS = 2, 16
NW = SC_NC * SC_NS
GCH = 32


def _router_body(l_ref, topi_ref, gates_ref):
    logits = l_ref[...]
    idx = lax.broadcasted_iota(jnp.int32, logits.shape, 1)
    m1 = jnp.max(logits, axis=1, keepdims=True)
    i1 = jnp.min(jnp.where(logits == m1, idx, E), axis=1, keepdims=True)
    l2 = jnp.where(idx == i1, -jnp.inf, logits)
    m2 = jnp.max(l2, axis=1, keepdims=True)
    i2 = jnp.min(jnp.where(l2 == m2, idx, E), axis=1, keepdims=True)
    e2 = jnp.exp(m2 - m1)
    denom = 1.0 + e2
    topi_ref[...] = jnp.concatenate([i1, i2], axis=1)
    gates_ref[...] = jnp.concatenate([1.0 / denom, e2 / denom], axis=1)


def _router(logits):
    return pl.pallas_call(
        _router_body,
        grid=(S // TMR,),
        in_specs=[
            pl.BlockSpec((TMR, E), lambda i: (i, 0)),
        ],
        out_specs=[
            pl.BlockSpec((TMR, K), lambda i: (i, 0)),
            pl.BlockSpec((TMR, K), lambda i: (i, 0)),
        ],
        out_shape=[
            jax.ShapeDtypeStruct((S, K), jnp.int32),
            jax.ShapeDtypeStruct((S, K), jnp.float32),
        ],
    )(logits)


def _route_metadata(topi):
    e_flat = topi.reshape(N).astype(jnp.int32)
    oneh = (e_flat[:, None] == jnp.arange(E, dtype=jnp.int32)[None, :])
    onehi = oneh.astype(jnp.int32)
    csum = jnp.cumsum(onehi, axis=0)
    counts = csum[-1]
    off = jnp.concatenate(
        [jnp.zeros((1,), jnp.int32), jnp.cumsum(counts)]).astype(jnp.int32)
    within = jnp.sum(csum * onehi, axis=1) - 1
    off_own = jnp.sum(off[None, :E] * onehi, axis=1)
    pos = off_own + within

    t_lo = off[:E] // TM
    t_hi = (off[1:] + TM - 1) // TM
    n_e = jnp.where(counts > 0, t_hi - t_lo, 0).astype(jnp.int32)
    cumu = jnp.cumsum(n_e)
    V = cumu[E - 1]
    v = jnp.arange(U, dtype=jnp.int32)
    ev = jnp.sum((v[:, None] >= cumu[None, :]).astype(jnp.int32), axis=1)
    real = v < V
    e_last = jnp.max(jnp.where(real, ev, -1))
    ev = jnp.where(real, jnp.minimum(ev, E - 1), e_last)
    sel = (ev[:, None] == jnp.arange(E, dtype=jnp.int32)[None, :]).astype(jnp.int32)
    t_lo_v = jnp.sum(sel * t_lo[None, :], axis=1)
    cumu_lo_v = jnp.sum(sel * (cumu - n_e)[None, :], axis=1)
    off_lo_v = jnp.sum(sel * off[None, :E], axis=1)
    off_hi_v = jnp.sum(sel * off[None, 1:], axis=1)
    ut = jnp.where(real, t_lo_v + (v - cumu_lo_v), NT - 1)
    us = jnp.where(real, jnp.maximum(off_lo_v, ut * TM), 0)
    ue = jnp.where(real, jnp.minimum(off_hi_v, (ut + 1) * TM), 0)
    uf = jnp.concatenate(
        [jnp.ones((1,), jnp.int32), (ut[1:] != ut[:-1]).astype(jnp.int32)])
    return pos, ev, ut, us, ue, uf


def _sc_mesh():
    return plsc.VectorSubcoreMesh(core_axis_name="c", subcore_axis_name="s",
                                  num_cores=SC_NC, num_subcores=SC_NS)


def _sc_gather_body(n_rows, table_hbm, idx_hbm, out_hbm, idx_v, rows_v, sem):
    bpw = n_rows // NW
    wid = lax.axis_index("s") * SC_NC + lax.axis_index("c")
    base = wid * bpw

    def chunk(i, carry):
        cb = base + i * GCH
        pltpu.sync_copy(idx_hbm.at[pl.ds(cb, GCH)], idx_v)
        pltpu.async_copy(table_hbm.at[idx_v], rows_v, sem).wait()
        pltpu.sync_copy(rows_v, out_hbm.at[pl.ds(cb, GCH)])
        return carry

    lax.fori_loop(0, bpw // GCH, chunk, 0)


def _sc_gather(table, idx, n_rows):
    return pl.kernel(
        functools.partial(_sc_gather_body, n_rows),
        out_type=jax.ShapeDtypeStruct((n_rows, D), jnp.float32),
        mesh=_sc_mesh(),
        scratch_types=[
            pltpu.VMEM((GCH,), jnp.int32),
            pltpu.VMEM((GCH, D), jnp.float32),
            pltpu.SemaphoreType.DMA,
        ],
    )(table, idx)


def _sc_dispatch_body(xf_hbm, tok_hbm, pos_hbm, out_hbm,
                      tok_v, pos_v, rows_v, sem):
    bpw = N // NW
    wid = lax.axis_index("s") * SC_NC + lax.axis_index("c")
    base = wid * bpw

    def chunk(i, carry):
        cb = base + i * GCH
        pltpu.sync_copy(tok_hbm.at[pl.ds(cb, GCH)], tok_v)
        pltpu.sync_copy(pos_hbm.at[pl.ds(cb, GCH)], pos_v)
        pltpu.async_copy(xf_hbm.at[tok_v], rows_v, sem).wait()
        pltpu.async_copy(rows_v, out_hbm.at[pos_v], sem).wait()
        return carry

    lax.fori_loop(0, bpw // GCH, chunk, 0)


def _sc_dispatch(xf, tok, pos):
    return pl.kernel(
        _sc_dispatch_body,
        out_type=jax.ShapeDtypeStruct((N, D), jnp.float32),
        mesh=_sc_mesh(),
        scratch_types=[
            pltpu.VMEM((GCH,), jnp.int32),
            pltpu.VMEM((GCH,), jnp.int32),
            pltpu.VMEM((GCH, D), jnp.float32),
            pltpu.SemaphoreType.DMA,
        ],
    )(xf, tok, pos)


def _ffn_body(ev_r, ut_r, us_r, ue_r, uf_r,
              x_ref, w1_ref, b1_ref, w2_ref, b2_ref, out_ref):
    u = pl.program_id(0)
    xb = x_ref[...].astype(jnp.bfloat16)
    h = jnp.dot(xb, w1_ref[0], preferred_element_type=jnp.float32)
    h = h + b1_ref[0, 0][None, :]
    a = jnp.maximum(h, 0.0).astype(jnp.bfloat16)
    o = jnp.dot(a, w2_ref[0], preferred_element_type=jnp.float32)
    o = o + b2_ref[0, 0][None, :]
    rows = ut_r[u] * TM + lax.broadcasted_iota(jnp.int32, (TM, 1), 0)
    m = (rows >= us_r[u]) & (rows < ue_r[u])
    contrib = jnp.where(m, o, 0.0)

    @pl.when(uf_r[u] == 1)
    def _init():
        out_ref[...] = contrib

    @pl.when(uf_r[u] == 0)
    def _accum():
        out_ref[...] = out_ref[...] + contrib


def _grouped_ffn(x_sorted, W1b, b1r, W2b, b2r, ev, ut, us, ue, uf):
    grid_spec = pltpu.PrefetchScalarGridSpec(
        num_scalar_prefetch=5,
        grid=(U,),
        in_specs=[
            pl.BlockSpec((TM, D), lambda u, ev, ut, us, ue, uf: (ut[u], 0)),
            pl.BlockSpec((1, D, F), lambda u, ev, ut, us, ue, uf: (ev[u], 0, 0)),
            pl.BlockSpec((1, 1, F), lambda u, ev, ut, us, ue, uf: (ev[u], 0, 0)),
            pl.BlockSpec((1, F, D), lambda u, ev, ut, us, ue, uf: (ev[u], 0, 0)),
            pl.BlockSpec((1, 1, D), lambda u, ev, ut, us, ue, uf: (ev[u], 0, 0)),
        ],
        out_specs=pl.BlockSpec((TM, D), lambda u, ev, ut, us, ue, uf: (ut[u], 0)),
    )
    return pl.pallas_call(
        _ffn_body,
        grid_spec=grid_spec,
        out_shape=jax.ShapeDtypeStruct((N, D), jnp.float32),
        compiler_params=pltpu.CompilerParams(
            dimension_semantics=("arbitrary",)),
    )(ev, ut, us, ue, uf, x_sorted, W1b, b1r, W2b, b2r)


def _pairsum_body(o_ref, g_ref, y_ref):
    y_ref[...] = (o_ref[:, :D] * g_ref[:, 0:1] +
                  o_ref[:, D:] * g_ref[:, 1:2])


def _pairsum(o_pair, gates):
    return pl.pallas_call(
        _pairsum_body,
        grid=(S // TMP,),
        in_specs=[
            pl.BlockSpec((TMP, 2 * D), lambda i: (i, 0)),
            pl.BlockSpec((TMP, K), lambda i: (i, 0)),
        ],
        out_specs=pl.BlockSpec((TMP, D), lambda i: (i, 0)),
        out_shape=jax.ShapeDtypeStruct((S, D), jnp.float32),
    )(o_pair, gates)


def kernel(x, Wg, W1, b1, W2, b2):
    xf = x.reshape(S, D)
    logits = xf @ Wg
    topi, gates = _router(logits)
    pos, ev, ut, us, ue, uf = _route_metadata(topi)

    tok = jnp.arange(N, dtype=jnp.int32) // K
    x_sorted = _sc_dispatch(xf, tok, pos)

    W1b = W1.astype(jnp.bfloat16)
    W2b = W2.astype(jnp.bfloat16)
    b1r = b1[:, None, :]
    b2r = b2[:, None, :]
    o_sorted = _grouped_ffn(x_sorted, W1b, b1r, W2b, b2r, ev, ut, us, ue, uf)

    o_pair = _sc_gather(o_sorted, pos, N)
    y = _pairsum(o_pair.reshape(S, 2 * D), gates)
    return y.reshape(B, T, D)


# speedup vs baseline: 1.3195x; 1.1818x over previous
import functools

import jax
import jax.numpy as jnp
from jax import lax
from jax.experimental import pallas as pl
from jax.experimental.pallas import tpu as pltpu
from jax.experimental.pallas import tpu_sc as plsc

B, T, D, E, F, K = 1, 2048, 1024, 8, 2048, 2
S = B * T
N = S * K
TM = 256
NT = N // TM
U = NT + E - 1
TMR = 512
TMP = 256

SC_NC, SC_NS = 2, 16
NW = SC_NC * SC_NS
GCH = 32


def _router_body(l_ref, topi_ref, gates_ref):
    logits = l_ref[...]
    idx = lax.broadcasted_iota(jnp.int32, logits.shape, 1)
    m1 = jnp.max(logits, axis=1, keepdims=True)
    i1 = jnp.min(jnp.where(logits == m1, idx, E), axis=1, keepdims=True)
    l2 = jnp.where(idx == i1, -jnp.inf, logits)
    m2 = jnp.max(l2, axis=1, keepdims=True)
    i2 = jnp.min(jnp.where(l2 == m2, idx, E), axis=1, keepdims=True)
    e2 = jnp.exp(m2 - m1)
    denom = 1.0 + e2
    topi_ref[...] = jnp.concatenate([i1, i2], axis=1)
    gates_ref[...] = jnp.concatenate([1.0 / denom, e2 / denom], axis=1)


def _router(logits):
    return pl.pallas_call(
        _router_body,
        grid=(S // TMR,),
        in_specs=[
            pl.BlockSpec((TMR, E), lambda i: (i, 0)),
        ],
        out_specs=[
            pl.BlockSpec((TMR, K), lambda i: (i, 0)),
            pl.BlockSpec((TMR, K), lambda i: (i, 0)),
        ],
        out_shape=[
            jax.ShapeDtypeStruct((S, K), jnp.int32),
            jax.ShapeDtypeStruct((S, K), jnp.float32),
        ],
    )(logits)


def _route_metadata(topi):
    e_flat = topi.reshape(N).astype(jnp.int32)
    oneh = (e_flat[:, None] == jnp.arange(E, dtype=jnp.int32)[None, :])
    onehi = oneh.astype(jnp.int32)
    csum = jnp.cumsum(onehi, axis=0)
    counts = csum[-1]
    off = jnp.concatenate(
        [jnp.zeros((1,), jnp.int32), jnp.cumsum(counts)]).astype(jnp.int32)
    within = jnp.sum(csum * onehi, axis=1) - 1
    off_own = jnp.sum(off[None, :E] * onehi, axis=1)
    pos = off_own + within

    t_lo = off[:E] // TM
    t_hi = (off[1:] + TM - 1) // TM
    n_e = jnp.where(counts > 0, t_hi - t_lo, 0).astype(jnp.int32)
    cumu = jnp.cumsum(n_e)
    V = cumu[E - 1]
    v = jnp.arange(U, dtype=jnp.int32)
    ev = jnp.sum((v[:, None] >= cumu[None, :]).astype(jnp.int32), axis=1)
    real = v < V
    e_last = jnp.max(jnp.where(real, ev, -1))
    ev = jnp.where(real, jnp.minimum(ev, E - 1), e_last)
    sel = (ev[:, None] == jnp.arange(E, dtype=jnp.int32)[None, :]).astype(jnp.int32)
    t_lo_v = jnp.sum(sel * t_lo[None, :], axis=1)
    cumu_lo_v = jnp.sum(sel * (cumu - n_e)[None, :], axis=1)
    off_lo_v = jnp.sum(sel * off[None, :E], axis=1)
    off_hi_v = jnp.sum(sel * off[None, 1:], axis=1)
    ut = jnp.where(real, t_lo_v + (v - cumu_lo_v), NT - 1)
    us = jnp.where(real, jnp.maximum(off_lo_v, ut * TM), 0)
    ue = jnp.where(real, jnp.minimum(off_hi_v, (ut + 1) * TM), 0)
    uf = jnp.concatenate(
        [jnp.ones((1,), jnp.int32), (ut[1:] != ut[:-1]).astype(jnp.int32)])
    return pos, ev, ut, us, ue, uf


def _sc_mesh():
    return plsc.VectorSubcoreMesh(core_axis_name="c", subcore_axis_name="s",
                                  num_cores=SC_NC, num_subcores=SC_NS)


def _sc_gather_body(n_rows, table_hbm, idx_hbm, out_hbm, idx_v, rows_v, sem):
    bpw = n_rows // NW
    wid = lax.axis_index("s") * SC_NC + lax.axis_index("c")
    base = wid * bpw

    def chunk(i, carry):
        cb = base + i * GCH
        pltpu.sync_copy(idx_hbm.at[pl.ds(cb, GCH)], idx_v)
        pltpu.async_copy(table_hbm.at[idx_v], rows_v, sem).wait()
        pltpu.sync_copy(rows_v, out_hbm.at[pl.ds(cb, GCH)])
        return carry

    lax.fori_loop(0, bpw // GCH, chunk, 0)


def _sc_gather(table, idx, n_rows):
    return pl.kernel(
        functools.partial(_sc_gather_body, n_rows),
        out_type=jax.ShapeDtypeStruct((n_rows, D), jnp.float32),
        mesh=_sc_mesh(),
        scratch_types=[
            pltpu.VMEM((GCH,), jnp.int32),
            pltpu.VMEM((GCH, D), jnp.float32),
            pltpu.SemaphoreType.DMA,
        ],
    )(table, idx)


def _sc_dispatch_body(xf_hbm, tok_hbm, pos_hbm, out_hbm,
                      tok_v, pos_v, rows_v, sem):
    bpw = N // NW
    wid = lax.axis_index("s") * SC_NC + lax.axis_index("c")
    base = wid * bpw

    def chunk(i, carry):
        cb = base + i * GCH
        pltpu.sync_copy(tok_hbm.at[pl.ds(cb, GCH)], tok_v)
        pltpu.sync_copy(pos_hbm.at[pl.ds(cb, GCH)], pos_v)
        pltpu.async_copy(xf_hbm.at[tok_v], rows_v, sem).wait()
        pltpu.async_copy(rows_v, out_hbm.at[pos_v], sem).wait()
        return carry

    lax.fori_loop(0, bpw // GCH, chunk, 0)


def _sc_dispatch(xf, tok, pos):
    return pl.kernel(
        _sc_dispatch_body,
        out_type=jax.ShapeDtypeStruct((N, D), jnp.float32),
        mesh=_sc_mesh(),
        scratch_types=[
            pltpu.VMEM((GCH,), jnp.int32),
            pltpu.VMEM((GCH,), jnp.int32),
            pltpu.VMEM((GCH, D), jnp.float32),
            pltpu.SemaphoreType.DMA,
        ],
    )(xf, tok, pos)


def _ffn_body(ev_r, ut_r, us_r, ue_r, uf_r,
              x_ref, w1_ref, b1_ref, w2_ref, b2_ref, out_ref,
              w1b_s, w2b_s):
    u = pl.program_id(0)
    prev_e = ev_r[jnp.maximum(u - 1, 0)]

    @pl.when((u == 0) | (ev_r[u] != prev_e))
    def _cast_weights():
        w1b_s[...] = w1_ref[0].astype(jnp.bfloat16)
        w2b_s[...] = w2_ref[0].astype(jnp.bfloat16)

    xb = x_ref[...].astype(jnp.bfloat16)
    h = jnp.dot(xb, w1b_s[...], preferred_element_type=jnp.float32)
    h = h + b1_ref[0, 0][None, :]
    a = jnp.maximum(h, 0.0).astype(jnp.bfloat16)
    o = jnp.dot(a, w2b_s[...], preferred_element_type=jnp.float32)
    o = o + b2_ref[0, 0][None, :]
    rows = ut_r[u] * TM + lax.broadcasted_iota(jnp.int32, (TM, 1), 0)
    m = (rows >= us_r[u]) & (rows < ue_r[u])
    contrib = jnp.where(m, o, 0.0)

    @pl.when(uf_r[u] == 1)
    def _init():
        out_ref[...] = contrib

    @pl.when(uf_r[u] == 0)
    def _accum():
        out_ref[...] = out_ref[...] + contrib


def _grouped_ffn(x_sorted, W1b, b1r, W2b, b2r, ev, ut, us, ue, uf):
    grid_spec = pltpu.PrefetchScalarGridSpec(
        num_scalar_prefetch=5,
        grid=(U,),
        in_specs=[
            pl.BlockSpec((TM, D), lambda u, ev, ut, us, ue, uf: (ut[u], 0)),
            pl.BlockSpec((1, D, F), lambda u, ev, ut, us, ue, uf: (ev[u], 0, 0)),
            pl.BlockSpec((1, 1, F), lambda u, ev, ut, us, ue, uf: (ev[u], 0, 0)),
            pl.BlockSpec((1, F, D), lambda u, ev, ut, us, ue, uf: (ev[u], 0, 0)),
            pl.BlockSpec((1, 1, D), lambda u, ev, ut, us, ue, uf: (ev[u], 0, 0)),
        ],
        out_specs=pl.BlockSpec((TM, D), lambda u, ev, ut, us, ue, uf: (ut[u], 0)),
        scratch_shapes=[
            pltpu.VMEM((D, F), jnp.bfloat16),
            pltpu.VMEM((F, D), jnp.bfloat16),
        ],
    )
    return pl.pallas_call(
        _ffn_body,
        grid_spec=grid_spec,
        out_shape=jax.ShapeDtypeStruct((N, D), jnp.float32),
        compiler_params=pltpu.CompilerParams(
            dimension_semantics=("arbitrary",)),
    )(ev, ut, us, ue, uf, x_sorted, W1b, b1r, W2b, b2r)


def _pairsum_body(o_ref, g_ref, y_ref):
    y_ref[...] = (o_ref[:, :D] * g_ref[:, 0:1] +
                  o_ref[:, D:] * g_ref[:, 1:2])


def _pairsum(o_pair, gates):
    return pl.pallas_call(
        _pairsum_body,
        grid=(S // TMP,),
        in_specs=[
            pl.BlockSpec((TMP, 2 * D), lambda i: (i, 0)),
            pl.BlockSpec((TMP, K), lambda i: (i, 0)),
        ],
        out_specs=pl.BlockSpec((TMP, D), lambda i: (i, 0)),
        out_shape=jax.ShapeDtypeStruct((S, D), jnp.float32),
    )(o_pair, gates)


def kernel(x, Wg, W1, b1, W2, b2):
    xf = x.reshape(S, D)
    logits = xf @ Wg
    topi, gates = _router(logits)
    pos, ev, ut, us, ue, uf = _route_metadata(topi)

    tok = jnp.arange(N, dtype=jnp.int32) // K
    x_sorted = _sc_dispatch(xf, tok, pos)

    b1r = b1[:, None, :]
    b2r = b2[:, None, :]
    o_sorted = _grouped_ffn(x_sorted, W1, b1r, W2, b2r, ev, ut, us, ue, uf)

    o_pair = _sc_gather(o_sorted, pos, N)
    y = _pairsum(o_pair.reshape(S, 2 * D), gates)
    return y.reshape(B, T, D)


# speedup vs baseline: 1.4655x; 1.1107x over previous
import functools

import jax
import jax.numpy as jnp
from jax import lax
from jax.experimental import pallas as pl
from jax.experimental.pallas import tpu as pltpu
from jax.experimental.pallas import tpu_sc as plsc

B, T, D, E, F, K = 1, 2048, 1024, 8, 2048, 2
S = B * T
N = S * K
TM = 512
NT = N // TM
U = NT + E - 1
TMR = 512
TMP = 256

SC_NC, SC_NS = 2, 16
NW = SC_NC * SC_NS
GCH = 32


def _router_body(l_ref, topi_ref, gates_ref):
    logits = l_ref[...]
    idx = lax.broadcasted_iota(jnp.int32, logits.shape, 1)
    m1 = jnp.max(logits, axis=1, keepdims=True)
    i1 = jnp.min(jnp.where(logits == m1, idx, E), axis=1, keepdims=True)
    l2 = jnp.where(idx == i1, -jnp.inf, logits)
    m2 = jnp.max(l2, axis=1, keepdims=True)
    i2 = jnp.min(jnp.where(l2 == m2, idx, E), axis=1, keepdims=True)
    e2 = jnp.exp(m2 - m1)
    denom = 1.0 + e2
    topi_ref[...] = jnp.concatenate([i1, i2], axis=1)
    gates_ref[...] = jnp.concatenate([1.0 / denom, e2 / denom], axis=1)


def _router(logits):
    return pl.pallas_call(
        _router_body,
        grid=(S // TMR,),
        in_specs=[
            pl.BlockSpec((TMR, E), lambda i: (i, 0)),
        ],
        out_specs=[
            pl.BlockSpec((TMR, K), lambda i: (i, 0)),
            pl.BlockSpec((TMR, K), lambda i: (i, 0)),
        ],
        out_shape=[
            jax.ShapeDtypeStruct((S, K), jnp.int32),
            jax.ShapeDtypeStruct((S, K), jnp.float32),
        ],
    )(logits)


def _route_metadata(topi):
    e_flat = topi.reshape(N).astype(jnp.int32)
    oneh = (e_flat[:, None] == jnp.arange(E, dtype=jnp.int32)[None, :])
    onehi = oneh.astype(jnp.int32)
    csum = jnp.cumsum(onehi, axis=0)
    counts = csum[-1]
    off = jnp.concatenate(
        [jnp.zeros((1,), jnp.int32), jnp.cumsum(counts)]).astype(jnp.int32)
    within = jnp.sum(csum * onehi, axis=1) - 1
    off_own = jnp.sum(off[None, :E] * onehi, axis=1)
    pos = off_own + within

    t_lo = off[:E] // TM
    t_hi = (off[1:] + TM - 1) // TM
    n_e = jnp.where(counts > 0, t_hi - t_lo, 0).astype(jnp.int32)
    cumu = jnp.cumsum(n_e)
    V = cumu[E - 1]
    v = jnp.arange(U, dtype=jnp.int32)
    ev = jnp.sum((v[:, None] >= cumu[None, :]).astype(jnp.int32), axis=1)
    real = v < V
    e_last = jnp.max(jnp.where(real, ev, -1))
    ev = jnp.where(real, jnp.minimum(ev, E - 1), e_last)
    sel = (ev[:, None] == jnp.arange(E, dtype=jnp.int32)[None, :]).astype(jnp.int32)
    t_lo_v = jnp.sum(sel * t_lo[None, :], axis=1)
    cumu_lo_v = jnp.sum(sel * (cumu - n_e)[None, :], axis=1)
    off_lo_v = jnp.sum(sel * off[None, :E], axis=1)
    off_hi_v = jnp.sum(sel * off[None, 1:], axis=1)
    ut = jnp.where(real, t_lo_v + (v - cumu_lo_v), NT - 1)
    us = jnp.where(real, jnp.maximum(off_lo_v, ut * TM), 0)
    ue = jnp.where(real, jnp.minimum(off_hi_v, (ut + 1) * TM), 0)
    uf = jnp.concatenate(
        [jnp.ones((1,), jnp.int32), (ut[1:] != ut[:-1]).astype(jnp.int32)])
    return pos, ev, ut, us, ue, uf


def _sc_mesh():
    return plsc.VectorSubcoreMesh(core_axis_name="c", subcore_axis_name="s",
                                  num_cores=SC_NC, num_subcores=SC_NS)


def _sc_gather_body(n_rows, table_hbm, idx_hbm, out_hbm, idx_v, rows_v, sem):
    bpw = n_rows // NW
    wid = lax.axis_index("s") * SC_NC + lax.axis_index("c")
    base = wid * bpw

    def chunk(i, carry):
        cb = base + i * GCH
        pltpu.sync_copy(idx_hbm.at[pl.ds(cb, GCH)], idx_v)
        pltpu.async_copy(table_hbm.at[idx_v], rows_v, sem).wait()
        pltpu.sync_copy(rows_v, out_hbm.at[pl.ds(cb, GCH)])
        return carry

    lax.fori_loop(0, bpw // GCH, chunk, 0)


def _sc_gather(table, idx, n_rows):
    return pl.kernel(
        functools.partial(_sc_gather_body, n_rows),
        out_type=jax.ShapeDtypeStruct((n_rows, D), jnp.float32),
        mesh=_sc_mesh(),
        scratch_types=[
            pltpu.VMEM((GCH,), jnp.int32),
            pltpu.VMEM((GCH, D), jnp.float32),
            pltpu.SemaphoreType.DMA,
        ],
    )(table, idx)


def _sc_dispatch_body(xf_hbm, tok_hbm, pos_hbm, out_hbm,
                      tok_v, pos_v, rows_v, sem):
    bpw = N // NW
    wid = lax.axis_index("s") * SC_NC + lax.axis_index("c")
    base = wid * bpw

    def chunk(i, carry):
        cb = base + i * GCH
        pltpu.sync_copy(tok_hbm.at[pl.ds(cb, GCH)], tok_v)
        pltpu.sync_copy(pos_hbm.at[pl.ds(cb, GCH)], pos_v)
        pltpu.async_copy(xf_hbm.at[tok_v], rows_v, sem).wait()
        pltpu.async_copy(rows_v, out_hbm.at[pos_v], sem).wait()
        return carry

    lax.fori_loop(0, bpw // GCH, chunk, 0)


def _sc_dispatch(xf, tok, pos):
    return pl.kernel(
        _sc_dispatch_body,
        out_type=jax.ShapeDtypeStruct((N, D), jnp.float32),
        mesh=_sc_mesh(),
        scratch_types=[
            pltpu.VMEM((GCH,), jnp.int32),
            pltpu.VMEM((GCH,), jnp.int32),
            pltpu.VMEM((GCH, D), jnp.float32),
            pltpu.SemaphoreType.DMA,
        ],
    )(xf, tok, pos)


def _ffn_body(ev_r, ut_r, us_r, ue_r, uf_r,
              x_ref, w1_ref, b1_ref, w2_ref, b2_ref, out_ref,
              w1b_s, w2b_s):
    u = pl.program_id(0)
    prev_e = ev_r[jnp.maximum(u - 1, 0)]

    @pl.when((u == 0) | (ev_r[u] != prev_e))
    def _cast_weights():
        w1b_s[...] = w1_ref[0].astype(jnp.bfloat16)
        w2b_s[...] = w2_ref[0].astype(jnp.bfloat16)

    xb = x_ref[...].astype(jnp.bfloat16)
    h = jnp.dot(xb, w1b_s[...], preferred_element_type=jnp.float32)
    h = h + b1_ref[0, 0][None, :]
    a = jnp.maximum(h, 0.0).astype(jnp.bfloat16)
    o = jnp.dot(a, w2b_s[...], preferred_element_type=jnp.float32)
    o = o + b2_ref[0, 0][None, :]
    rows = ut_r[u] * TM + lax.broadcasted_iota(jnp.int32, (TM, 1), 0)
    m = (rows >= us_r[u]) & (rows < ue_r[u])
    contrib = jnp.where(m, o, 0.0)

    @pl.when(uf_r[u] == 1)
    def _init():
        out_ref[...] = contrib

    @pl.when(uf_r[u] == 0)
    def _accum():
        out_ref[...] = out_ref[...] + contrib


def _grouped_ffn(x_sorted, W1b, b1r, W2b, b2r, ev, ut, us, ue, uf):
    grid_spec = pltpu.PrefetchScalarGridSpec(
        num_scalar_prefetch=5,
        grid=(U,),
        in_specs=[
            pl.BlockSpec((TM, D), lambda u, ev, ut, us, ue, uf: (ut[u], 0)),
            pl.BlockSpec((1, D, F), lambda u, ev, ut, us, ue, uf: (ev[u], 0, 0)),
            pl.BlockSpec((1, 1, F), lambda u, ev, ut, us, ue, uf: (ev[u], 0, 0)),
            pl.BlockSpec((1, F, D), lambda u, ev, ut, us, ue, uf: (ev[u], 0, 0)),
            pl.BlockSpec((1, 1, D), lambda u, ev, ut, us, ue, uf: (ev[u], 0, 0)),
        ],
        out_specs=pl.BlockSpec((TM, D), lambda u, ev, ut, us, ue, uf: (ut[u], 0)),
        scratch_shapes=[
            pltpu.VMEM((D, F), jnp.bfloat16),
            pltpu.VMEM((F, D), jnp.bfloat16),
        ],
    )
    return pl.pallas_call(
        _ffn_body,
        grid_spec=grid_spec,
        out_shape=jax.ShapeDtypeStruct((N, D), jnp.float32),
        compiler_params=pltpu.CompilerParams(
            dimension_semantics=("arbitrary",)),
    )(ev, ut, us, ue, uf, x_sorted, W1b, b1r, W2b, b2r)


def _pairsum_body(o_ref, g_ref, y_ref):
    o = o_ref[...].reshape(TMP, 2, D)
    y_ref[...] = (o[:, 0, :] * g_ref[:, 0:1] + o[:, 1, :] * g_ref[:, 1:2])


def _pairsum(o_pair, gates):
    return pl.pallas_call(
        _pairsum_body,
        grid=(S // TMP,),
        in_specs=[
            pl.BlockSpec((2 * TMP, D), lambda i: (i, 0)),
            pl.BlockSpec((TMP, K), lambda i: (i, 0)),
        ],
        out_specs=pl.BlockSpec((TMP, D), lambda i: (i, 0)),
        out_shape=jax.ShapeDtypeStruct((S, D), jnp.float32),
    )(o_pair, gates)


def kernel(x, Wg, W1, b1, W2, b2):
    xf = x.reshape(S, D)
    logits = xf @ Wg
    topi, gates = _router(logits)
    pos, ev, ut, us, ue, uf = _route_metadata(topi)

    tok = jnp.arange(N, dtype=jnp.int32) // K
    x_sorted = _sc_dispatch(xf, tok, pos)

    b1r = b1[:, None, :]
    b2r = b2[:, None, :]
    o_sorted = _grouped_ffn(x_sorted, W1, b1r, W2, b2r, ev, ut, us, ue, uf)

    o_pair = _sc_gather(o_sorted, pos, N)
    y = _pairsum(o_pair, gates)
    return y.reshape(B, T, D)


# speedup vs baseline: 1.5491x; 1.0571x over previous
import functools

import jax
import jax.numpy as jnp
from jax import lax
from jax.experimental import pallas as pl
from jax.experimental.pallas import tpu as pltpu
from jax.experimental.pallas import tpu_sc as plsc

B, T, D, E, F, K = 1, 2048, 1024, 8, 2048, 2
S = B * T
N = S * K
TM = 512
NT = N // TM
U = NT + E - 1
TMR = 512
TMP = 256

SC_NC, SC_NS = 2, 16
NW = SC_NC * SC_NS
GCH = 32


def _router_body(l_ref, topi_ref, gates_ref):
    logits = l_ref[...]
    idx = lax.broadcasted_iota(jnp.int32, logits.shape, 1)
    m1 = jnp.max(logits, axis=1, keepdims=True)
    i1 = jnp.min(jnp.where(logits == m1, idx, E), axis=1, keepdims=True)
    l2 = jnp.where(idx == i1, -jnp.inf, logits)
    m2 = jnp.max(l2, axis=1, keepdims=True)
    i2 = jnp.min(jnp.where(l2 == m2, idx, E), axis=1, keepdims=True)
    e2 = jnp.exp(m2 - m1)
    denom = 1.0 + e2
    topi_ref[...] = jnp.concatenate([i1, i2], axis=1)
    gates_ref[...] = jnp.concatenate([1.0 / denom, e2 / denom], axis=1)


def _router(logits):
    return pl.pallas_call(
        _router_body,
        grid=(S // TMR,),
        in_specs=[
            pl.BlockSpec((TMR, E), lambda i: (i, 0)),
        ],
        out_specs=[
            pl.BlockSpec((TMR, K), lambda i: (i, 0)),
            pl.BlockSpec((TMR, K), lambda i: (i, 0)),
        ],
        out_shape=[
            jax.ShapeDtypeStruct((S, K), jnp.int32),
            jax.ShapeDtypeStruct((S, K), jnp.float32),
        ],
    )(logits)


def _route_metadata(topi):
    e_flat = topi.reshape(N).astype(jnp.int32)
    oneh = (e_flat[:, None] == jnp.arange(E, dtype=jnp.int32)[None, :])
    onehi = oneh.astype(jnp.int32)
    csum = jnp.cumsum(onehi, axis=0)
    counts = csum[-1]
    off = jnp.concatenate(
        [jnp.zeros((1,), jnp.int32), jnp.cumsum(counts)]).astype(jnp.int32)
    within = jnp.sum(csum * onehi, axis=1) - 1
    off_own = jnp.sum(off[None, :E] * onehi, axis=1)
    pos = off_own + within

    t_lo = off[:E] // TM
    t_hi = (off[1:] + TM - 1) // TM
    n_e = jnp.where(counts > 0, t_hi - t_lo, 0).astype(jnp.int32)
    cumu = jnp.cumsum(n_e)
    V = cumu[E - 1]
    v = jnp.arange(U, dtype=jnp.int32)
    ev = jnp.sum((v[:, None] >= cumu[None, :]).astype(jnp.int32), axis=1)
    real = v < V
    e_last = jnp.max(jnp.where(real, ev, -1))
    ev = jnp.where(real, jnp.minimum(ev, E - 1), e_last)
    sel = (ev[:, None] == jnp.arange(E, dtype=jnp.int32)[None, :]).astype(jnp.int32)
    t_lo_v = jnp.sum(sel * t_lo[None, :], axis=1)
    cumu_lo_v = jnp.sum(sel * (cumu - n_e)[None, :], axis=1)
    off_lo_v = jnp.sum(sel * off[None, :E], axis=1)
    off_hi_v = jnp.sum(sel * off[None, 1:], axis=1)
    ut = jnp.where(real, t_lo_v + (v - cumu_lo_v), NT - 1)
    us = jnp.where(real, jnp.maximum(off_lo_v, ut * TM), 0)
    ue = jnp.where(real, jnp.minimum(off_hi_v, (ut + 1) * TM), 0)
    uf = jnp.concatenate(
        [jnp.ones((1,), jnp.int32), (ut[1:] != ut[:-1]).astype(jnp.int32)])
    return pos, ev, ut, us, ue, uf


def _sc_mesh():
    return plsc.VectorSubcoreMesh(core_axis_name="c", subcore_axis_name="s",
                                  num_cores=SC_NC, num_subcores=SC_NS)


def _sc_dispatch_body(xf_hbm, tok_hbm, pos_hbm, out_hbm,
                      tok0, tok1, pos0, pos1, r0, r1,
                      gs0, gs1, ws0, ws1):
    bpw = N // NW
    nch = bpw // GCH
    wid = lax.axis_index("s") * SC_NC + lax.axis_index("c")
    base = wid * bpw
    toks, poss, rows = [tok0, tok1], [pos0, pos1], [r0, r1]
    gsem, wsem = [gs0, gs1], [ws0, ws1]
    gh, wb = [None, None], [None, None]

    pltpu.sync_copy(tok_hbm.at[pl.ds(base, GCH)], tok0)
    pltpu.sync_copy(pos_hbm.at[pl.ds(base, GCH)], pos0)
    gh[0] = pltpu.async_copy(xf_hbm.at[tok0], r0, gs0)
    for i in range(nch):
        b, nb = i % 2, (i + 1) % 2
        if i + 1 < nch:
            if wb[nb] is not None:
                wb[nb].wait()
                wb[nb] = None
            cb = base + (i + 1) * GCH
            pltpu.sync_copy(tok_hbm.at[pl.ds(cb, GCH)], toks[nb])
            pltpu.sync_copy(pos_hbm.at[pl.ds(cb, GCH)], poss[nb])
            gh[nb] = pltpu.async_copy(xf_hbm.at[toks[nb]], rows[nb], gsem[nb])
        gh[b].wait()
        wb[b] = pltpu.async_copy(rows[b], out_hbm.at[poss[b]], wsem[b])
    for b in range(2):
        if wb[b] is not None:
            wb[b].wait()


def _sc_dispatch(xf, tok, pos):
    return pl.kernel(
        _sc_dispatch_body,
        out_type=jax.ShapeDtypeStruct((N, D), jnp.float32),
        mesh=_sc_mesh(),
        scratch_types=[
            pltpu.VMEM((GCH,), jnp.int32),
            pltpu.VMEM((GCH,), jnp.int32),
            pltpu.VMEM((GCH,), jnp.int32),
            pltpu.VMEM((GCH,), jnp.int32),
            pltpu.VMEM((GCH, D), jnp.float32),
            pltpu.VMEM((GCH, D), jnp.float32),
            pltpu.SemaphoreType.DMA,
            pltpu.SemaphoreType.DMA,
            pltpu.SemaphoreType.DMA,
            pltpu.SemaphoreType.DMA,
        ],
    )(xf, tok, pos)


def _sc_combine_body(table_hbm, idxe_hbm, idxo_hbm, outa_hbm, outb_hbm,
                     i0, i1, r0, r1, gs0, gs1, ws0, ws1):
    tpw = S // NW
    wid = lax.axis_index("s") * SC_NC + lax.axis_index("c")
    tb = wid * tpw
    items = []
    for c in range(tpw // GCH):
        items.append((idxe_hbm, outa_hbm, tb + c * GCH))
        items.append((idxo_hbm, outb_hbm, tb + c * GCH))
    idxs, rows = [i0, i1], [r0, r1]
    gsem, wsem = [gs0, gs1], [ws0, ws1]
    gh, wb = [None, None], [None, None]

    src0, _, off0 = items[0]
    pltpu.sync_copy(src0.at[pl.ds(off0, GCH)], i0)
    gh[0] = pltpu.async_copy(table_hbm.at[i0], r0, gs0)
    for i in range(len(items)):
        b, nb = i % 2, (i + 1) % 2
        if i + 1 < len(items):
            if wb[nb] is not None:
                wb[nb].wait()
                wb[nb] = None
            srcn, _, offn = items[i + 1]
            pltpu.sync_copy(srcn.at[pl.ds(offn, GCH)], idxs[nb])
            gh[nb] = pltpu.async_copy(table_hbm.at[idxs[nb]], rows[nb], gsem[nb])
        gh[b].wait()
        _, out, off = items[i]
        wb[b] = pltpu.async_copy(rows[b], out.at[pl.ds(off, GCH)], wsem[b])
    for b in range(2):
        if wb[b] is not None:
            wb[b].wait()


def _sc_combine(table, idx_e, idx_o):
    return pl.kernel(
        _sc_combine_body,
        out_type=[
            jax.ShapeDtypeStruct((S, D), jnp.float32),
            jax.ShapeDtypeStruct((S, D), jnp.float32),
        ],
        mesh=_sc_mesh(),
        scratch_types=[
            pltpu.VMEM((GCH,), jnp.int32),
            pltpu.VMEM((GCH,), jnp.int32),
            pltpu.VMEM((GCH, D), jnp.float32),
            pltpu.VMEM((GCH, D), jnp.float32),
            pltpu.SemaphoreType.DMA,
            pltpu.SemaphoreType.DMA,
            pltpu.SemaphoreType.DMA,
            pltpu.SemaphoreType.DMA,
        ],
    )(table, idx_e, idx_o)


def _ffn_body(ev_r, ut_r, us_r, ue_r, uf_r,
              x_ref, w1_ref, b1_ref, w2_ref, b2_ref, out_ref,
              w1b_s, w2b_s):
    u = pl.program_id(0)
    prev_e = ev_r[jnp.maximum(u - 1, 0)]

    @pl.when((u == 0) | (ev_r[u] != prev_e))
    def _cast_weights():
        w1b_s[...] = w1_ref[0].astype(jnp.bfloat16)
        w2b_s[...] = w2_ref[0].astype(jnp.bfloat16)

    xb = x_ref[...].astype(jnp.bfloat16)
    h = jnp.dot(xb, w1b_s[...], preferred_element_type=jnp.float32)
    h = h + b1_ref[0, 0][None, :]
    a = jnp.maximum(h, 0.0).astype(jnp.bfloat16)
    o = jnp.dot(a, w2b_s[...], preferred_element_type=jnp.float32)
    o = o + b2_ref[0, 0][None, :]
    rows = ut_r[u] * TM + lax.broadcasted_iota(jnp.int32, (TM, 1), 0)
    m = (rows >= us_r[u]) & (rows < ue_r[u])
    contrib = jnp.where(m, o, 0.0)

    @pl.when(uf_r[u] == 1)
    def _init():
        out_ref[...] = contrib

    @pl.when(uf_r[u] == 0)
    def _accum():
        out_ref[...] = out_ref[...] + contrib


def _grouped_ffn(x_sorted, W1b, b1r, W2b, b2r, ev, ut, us, ue, uf):
    grid_spec = pltpu.PrefetchScalarGridSpec(
        num_scalar_prefetch=5,
        grid=(U,),
        in_specs=[
            pl.BlockSpec((TM, D), lambda u, ev, ut, us, ue, uf: (ut[u], 0)),
            pl.BlockSpec((1, D, F), lambda u, ev, ut, us, ue, uf: (ev[u], 0, 0)),
            pl.BlockSpec((1, 1, F), lambda u, ev, ut, us, ue, uf: (ev[u], 0, 0)),
            pl.BlockSpec((1, F, D), lambda u, ev, ut, us, ue, uf: (ev[u], 0, 0)),
            pl.BlockSpec((1, 1, D), lambda u, ev, ut, us, ue, uf: (ev[u], 0, 0)),
        ],
        out_specs=pl.BlockSpec((TM, D), lambda u, ev, ut, us, ue, uf: (ut[u], 0)),
        scratch_shapes=[
            pltpu.VMEM((D, F), jnp.bfloat16),
            pltpu.VMEM((F, D), jnp.bfloat16),
        ],
    )
    return pl.pallas_call(
        _ffn_body,
        grid_spec=grid_spec,
        out_shape=jax.ShapeDtypeStruct((N, D), jnp.float32),
        compiler_params=pltpu.CompilerParams(
            dimension_semantics=("arbitrary",)),
    )(ev, ut, us, ue, uf, x_sorted, W1b, b1r, W2b, b2r)


def _pairsum_body(a_ref, b_ref, g_ref, y_ref):
    y_ref[...] = (a_ref[...] * g_ref[:, 0:1] + b_ref[...] * g_ref[:, 1:2])


def _pairsum(o_a, o_b, gates):
    return pl.pallas_call(
        _pairsum_body,
        grid=(S // TMP,),
        in_specs=[
            pl.BlockSpec((TMP, D), lambda i: (i, 0)),
            pl.BlockSpec((TMP, D), lambda i: (i, 0)),
            pl.BlockSpec((TMP, K), lambda i: (i, 0)),
        ],
        out_specs=pl.BlockSpec((TMP, D), lambda i: (i, 0)),
        out_shape=jax.ShapeDtypeStruct((S, D), jnp.float32),
    )(o_a, o_b, gates)


def kernel(x, Wg, W1, b1, W2, b2):
    xf = x.reshape(S, D)
    logits = xf @ Wg
    topi, gates = _router(logits)
    pos, ev, ut, us, ue, uf = _route_metadata(topi)

    tok = jnp.arange(N, dtype=jnp.int32) // K
    x_sorted = _sc_dispatch(xf, tok, pos)

    b1r = b1[:, None, :]
    b2r = b2[:, None, :]
    o_sorted = _grouped_ffn(x_sorted, W1, b1r, W2, b2r, ev, ut, us, ue, uf)

    o_a, o_b = _sc_combine(o_sorted, pos[0::2], pos[1::2])
    y = _pairsum(o_a, o_b, gates)
    return y.reshape(B, T, D)
